# baseline (speedup 1.0000x reference)
import functools

import jax
import jax.numpy as jnp
from jax.experimental import pallas as pl
from jax.experimental.pallas import tpu as pltpu


def _head_body(x_ref, w_ref, b_ref, o_ref):
    o_ref[...] = (
        jnp.dot(x_ref[...], w_ref[...], preferred_element_type=jnp.float32)
        + b_ref[...]
    )


def _tail_body(x_ref, p_ref, w_ref, b_ref, o_ref, h_ref):
    @pl.when(pl.program_id(0) == 0)
    def _():
        h_ref[...] = jnp.dot(
            x_ref[...], p_ref[...], preferred_element_type=jnp.float32
        ).astype(jnp.bfloat16)

    o_ref[...] = (
        jnp.dot(h_ref[...], w_ref[...], preferred_element_type=jnp.float32)
        + b_ref[...]
    )


def _head_call(x, w, b, bn):
    n_tok, d = x.shape
    n_out = w.shape[1]
    grid = (pl.cdiv(n_out, bn),)
    return pl.pallas_call(
        _head_body,
        grid=grid,
        in_specs=[
            pl.BlockSpec((n_tok, d), lambda j: (0, 0)),
            pl.BlockSpec((d, bn), lambda j: (0, j)),
            pl.BlockSpec((1, bn), lambda j: (0, j)),
        ],
        out_specs=pl.BlockSpec((n_tok, bn), lambda j: (0, j)),
        out_shape=jax.ShapeDtypeStruct((n_tok, n_out), jnp.float32),
    )(x, w, b)


def _tail_call(x, p, w, b, bn):
    n_tok, d = x.shape
    k, n_out = w.shape
    grid = (pl.cdiv(n_out, bn),)
    return pl.pallas_call(
        _tail_body,
        grid=grid,
        in_specs=[
            pl.BlockSpec((n_tok, d), lambda j: (0, 0)),
            pl.BlockSpec((d, k), lambda j: (0, 0)),
            pl.BlockSpec((k, bn), lambda j: (0, j)),
            pl.BlockSpec((1, bn), lambda j: (0, j)),
        ],
        out_specs=pl.BlockSpec((n_tok, bn), lambda j: (0, j)),
        out_shape=jax.ShapeDtypeStruct((n_tok, n_out), jnp.float32),
        scratch_shapes=[pltpu.VMEM((n_tok, k), jnp.bfloat16)],
    )(x, p, w, b)


@functools.partial(jax.jit, static_argnums=())
def kernel(x, W0, b0, P1, W1, b1, P2, W2, b2):
    xb = x.astype(jnp.bfloat16)
    logits_head = _head_call(
        xb, W0.astype(jnp.bfloat16), b0.reshape(1, -1), bn=1024
    )
    logits_c1 = _tail_call(
        xb,
        P1.astype(jnp.bfloat16),
        W1.astype(jnp.bfloat16),
        b1.reshape(1, -1),
        bn=1024,
    )
    logits_c2 = _tail_call(
        xb,
        P2.astype(jnp.bfloat16),
        W2.astype(jnp.bfloat16),
        b2.reshape(1, -1),
        bn=1024,
    )
    return (logits_head, logits_c1, logits_c2)


# speedup vs baseline: 1.0032x; 1.0032x over previous
import functools

import jax
import jax.numpy as jnp
from jax.experimental import pallas as pl
from jax.experimental.pallas import tpu as pltpu


def _head_body(x_ref, w_ref, b_ref, o_ref):
    o_ref[...] = (
        jnp.dot(x_ref[...], w_ref[...], preferred_element_type=jnp.float32)
        + b_ref[...]
    )


def _tail_body(x_ref, p_ref, w_ref, b_ref, o_ref, h_ref):
    @pl.when(pl.program_id(0) == 0)
    def _():
        h_ref[...] = jnp.dot(
            x_ref[...], p_ref[...], preferred_element_type=jnp.float32
        ).astype(jnp.bfloat16)

    o_ref[...] = (
        jnp.dot(h_ref[...], w_ref[...], preferred_element_type=jnp.float32)
        + b_ref[...]
    )


def _head_call(x, w, b, bn):
    n_tok, d = x.shape
    n_out = w.shape[1]
    grid = (pl.cdiv(n_out, bn),)
    return pl.pallas_call(
        _head_body,
        grid=grid,
        in_specs=[
            pl.BlockSpec((n_tok, d), lambda j: (0, 0)),
            pl.BlockSpec((d, bn), lambda j: (0, j)),
            pl.BlockSpec((1, bn), lambda j: (0, j)),
        ],
        out_specs=pl.BlockSpec((n_tok, bn), lambda j: (0, j)),
        out_shape=jax.ShapeDtypeStruct((n_tok, n_out), jnp.float32),
    )(x, w, b)


def _tail_call(x, p, w, b, bn):
    n_tok, d = x.shape
    k, n_out = w.shape
    grid = (pl.cdiv(n_out, bn),)
    return pl.pallas_call(
        _tail_body,
        grid=grid,
        in_specs=[
            pl.BlockSpec((n_tok, d), lambda j: (0, 0)),
            pl.BlockSpec((d, k), lambda j: (0, 0)),
            pl.BlockSpec((k, bn), lambda j: (0, j)),
            pl.BlockSpec((1, bn), lambda j: (0, j)),
        ],
        out_specs=pl.BlockSpec((n_tok, bn), lambda j: (0, j)),
        out_shape=jax.ShapeDtypeStruct((n_tok, n_out), jnp.float32),
        scratch_shapes=[pltpu.VMEM((n_tok, k), jnp.bfloat16)],
    )(x, p, w, b)


@functools.partial(jax.jit, static_argnums=())
def kernel(x, W0, b0, P1, W1, b1, P2, W2, b2):
    xb = x.astype(jnp.bfloat16)
    logits_head = _head_call(
        xb, W0.astype(jnp.bfloat16), b0.reshape(1, -1), bn=1024
    )
    logits_c1 = _tail_call(
        xb,
        P1.astype(jnp.bfloat16),
        W1.astype(jnp.bfloat16),
        b1.reshape(1, -1),
        bn=1024,
    )
    logits_c2 = _tail_call(
        xb,
        P2.astype(jnp.bfloat16),
        W2.astype(jnp.bfloat16),
        b2.reshape(1, -1),
        bn=2048,
    )
    return (logits_head, logits_c1, logits_c2)


# speedup vs baseline: 1.0046x; 1.0014x over previous
import functools

import jax
import jax.numpy as jnp
from jax.experimental import pallas as pl
from jax.experimental.pallas import tpu as pltpu


def _cluster_body(nslot, has_proj, *refs):
    if has_proj:
        x_ref, p_ref, w_ref, b_ref, o_hbm, obuf, sems = refs
    else:
        x_ref, w_ref, b_ref, o_hbm, obuf, sems = refs
        p_ref = None
    j = pl.program_id(0)
    nsteps = pl.num_programs(0)
    bm = obuf.shape[1]
    slot = jax.lax.rem(j, nslot)

    @pl.when(j >= nslot)
    def _():
        pltpu.make_async_copy(
            obuf.at[slot],
            o_hbm.at[pl.ds((j - nslot) * bm, bm), :],
            sems.at[slot],
        ).wait()

    lhs = x_ref[...]
    if p_ref is not None:
        lhs = jnp.dot(
            lhs, p_ref[...], preferred_element_type=jnp.float32
        ).astype(jnp.bfloat16)
    obuf[slot] = (
        jnp.dot(lhs, w_ref[...], preferred_element_type=jnp.float32)
        + b_ref[...]
    )
    pltpu.make_async_copy(
        obuf.at[slot], o_hbm.at[pl.ds(j * bm, bm), :], sems.at[slot]
    ).start()

    @pl.when(j == nsteps - 1)
    def _():
        for k in range(nslot):
            sj = j - k

            @pl.when(sj >= 0)
            def _wait():
                pltpu.make_async_copy(
                    obuf.at[jax.lax.rem(sj, nslot)],
                    o_hbm.at[pl.ds(sj * bm, bm), :],
                    sems.at[jax.lax.rem(sj, nslot)],
                ).wait()


def _cluster_call(x, p, w, b, bm, nslot):
    n_tok, d = x.shape
    k, n_out = w.shape
    grid = (n_tok // bm,)
    has_proj = p is not None
    in_specs = [pl.BlockSpec((bm, d), lambda i: (i, 0))]
    args = [x]
    if has_proj:
        in_specs.append(pl.BlockSpec((d, k), lambda i: (0, 0)))
        args.append(p)
    in_specs += [
        pl.BlockSpec((k, n_out), lambda i: (0, 0)),
        pl.BlockSpec((1, n_out), lambda i: (0, 0)),
    ]
    args += [w, b]
    return pl.pallas_call(
        functools.partial(_cluster_body, nslot, has_proj),
        grid=grid,
        in_specs=in_specs,
        out_specs=pl.BlockSpec(memory_space=pltpu.MemorySpace.HBM),
        out_shape=jax.ShapeDtypeStruct((n_tok, n_out), jnp.float32),
        scratch_shapes=[
            pltpu.VMEM((nslot, bm, n_out), jnp.float32),
            pltpu.SemaphoreType.DMA((nslot,)),
        ],
    )(*args)


def kernel(x, W0, b0, P1, W1, b1, P2, W2, b2):
    xb = x.astype(jnp.bfloat16)
    logits_head = _cluster_call(
        xb, None, W0.astype(jnp.bfloat16), b0.reshape(1, -1), bm=512, nslot=4
    )
    logits_c1 = _cluster_call(
        xb,
        P1.astype(jnp.bfloat16),
        W1.astype(jnp.bfloat16),
        b1.reshape(1, -1),
        bm=256,
        nslot=4,
    )
    logits_c2 = _cluster_call(
        xb,
        P2.astype(jnp.bfloat16),
        W2.astype(jnp.bfloat16),
        b2.reshape(1, -1),
        bm=32,
        nslot=3,
    )
    return (logits_head, logits_c1, logits_c2)


# speedup vs baseline: 2.7400x; 2.7274x over previous
import functools

import jax
import jax.numpy as jnp
from jax import lax
from jax.experimental import pallas as pl
from jax.experimental.pallas import tpu as pltpu

_BF = jnp.bfloat16
_F32 = jnp.float32


def _xt_body(x_ref, o_ref):
    o_ref[...] = x_ref[...].astype(_BF).T


def _xt_call(x):
    n_tok, d = x.shape
    return pl.pallas_call(
        _xt_body,
        out_shape=jax.ShapeDtypeStruct((d, n_tok), _BF),
    )(x)


def _head_body(xt_ref, wt_ref, b_ref, o_ref):
    o_ref[...] = (
        jnp.dot(wt_ref[...].astype(_BF), xt_ref[...], preferred_element_type=_F32)
        + b_ref[...]
    )


def _tail_body(xt_ref, p_ref, w_ref, b_ref, o_ref, h_ref, *, w_transposed):
    @pl.when(pl.program_id(0) == 0)
    def _():
        h_ref[...] = lax.dot_general(
            p_ref[...].astype(_BF),
            xt_ref[...],
            (((0,), (0,)), ((), ())),
            preferred_element_type=_F32,
        ).astype(_BF)

    if w_transposed:
        acc = jnp.dot(w_ref[...].astype(_BF), h_ref[...], preferred_element_type=_F32)
    else:
        acc = lax.dot_general(
            w_ref[...].astype(_BF),
            h_ref[...],
            (((0,), (0,)), ((), ())),
            preferred_element_type=_F32,
        )
    o_ref[...] = acc + b_ref[...]


def _head_call(xt, wt, b, bn):
    d, n_tok = xt.shape
    n_out = wt.shape[0]
    return pl.pallas_call(
        _head_body,
        grid=(pl.cdiv(n_out, bn),),
        in_specs=[
            pl.BlockSpec((d, n_tok), lambda j: (0, 0)),
            pl.BlockSpec((bn, d), lambda j: (j, 0)),
            pl.BlockSpec((bn, 1), lambda j: (j, 0)),
        ],
        out_specs=pl.BlockSpec((bn, n_tok), lambda j: (j, 0)),
        out_shape=jax.ShapeDtypeStruct((n_out, n_tok), _F32),
    )(xt, wt, b)


def _tail_call(xt, p, w, b, bn, w_transposed):
    d, n_tok = xt.shape
    k = p.shape[1]
    n_out = w.shape[0] if w_transposed else w.shape[1]
    if w_transposed:
        w_spec = pl.BlockSpec((bn, k), lambda j: (j, 0))
    else:
        w_spec = pl.BlockSpec((k, bn), lambda j: (0, j))
    return pl.pallas_call(
        functools.partial(_tail_body, w_transposed=w_transposed),
        grid=(pl.cdiv(n_out, bn),),
        in_specs=[
            pl.BlockSpec((d, n_tok), lambda j: (0, 0)),
            pl.BlockSpec((d, k), lambda j: (0, 0)),
            w_spec,
            pl.BlockSpec((bn, 1), lambda j: (j, 0)),
        ],
        out_specs=pl.BlockSpec((bn, n_tok), lambda j: (j, 0)),
        out_shape=jax.ShapeDtypeStruct((n_out, n_tok), _F32),
        scratch_shapes=[pltpu.VMEM((k, n_tok), _BF)],
    )(xt, p, w, b)


def kernel(x, W0, b0, P1, W1, b1, P2, W2, b2):
    xt = _xt_call(x)
    lh = _head_call(xt, W0.T, b0.reshape(-1, 1), bn=512)
    lc1 = _tail_call(xt, P1, W1.T, b1.reshape(-1, 1), bn=512, w_transposed=True)
    lc2 = _tail_call(xt, P2, W2, b2.reshape(-1, 1), bn=512, w_transposed=False)
    return (lh.T, lc1.T, lc2.T)


# speedup vs baseline: 3.0794x; 1.1239x over previous
import functools

import jax
import jax.numpy as jnp
from jax import lax
from jax.experimental import pallas as pl
from jax.experimental.pallas import tpu as pltpu

_BF = jnp.bfloat16
_F32 = jnp.float32


def _xt_body(x_ref, o_ref):
    o_ref[...] = x_ref[...].astype(_BF).T


def _xt_call(x):
    n_tok, d = x.shape
    return pl.pallas_call(
        _xt_body,
        out_shape=jax.ShapeDtypeStruct((d, n_tok), _BF),
    )(x)


def _head_body(xt_ref, wt_ref, b_ref, o_ref):
    o_ref[...] = (
        jnp.dot(wt_ref[...].astype(_BF), xt_ref[...], preferred_element_type=_F32)
        + b_ref[...]
    )


def _tail_body(xt_ref, p_ref, w_ref, b_ref, o_ref, h_ref, *, w_transposed):
    @pl.when(pl.program_id(0) == 0)
    def _():
        h_ref[...] = lax.dot_general(
            p_ref[...].astype(_BF),
            xt_ref[...],
            (((0,), (0,)), ((), ())),
            preferred_element_type=_F32,
        ).astype(_BF)

    if w_transposed:
        acc = jnp.dot(w_ref[...].astype(_BF), h_ref[...], preferred_element_type=_F32)
    else:
        acc = lax.dot_general(
            w_ref[...].astype(_BF),
            h_ref[...],
            (((0,), (0,)), ((), ())),
            preferred_element_type=_F32,
        )
    o_ref[...] = acc + b_ref[...]


def _head_call(xt, wt, b, bn):
    d, n_tok = xt.shape
    n_out = wt.shape[0]
    return pl.pallas_call(
        _head_body,
        grid=(pl.cdiv(n_out, bn),),
        in_specs=[
            pl.BlockSpec((d, n_tok), lambda j: (0, 0)),
            pl.BlockSpec((bn, d), lambda j: (j, 0)),
            pl.BlockSpec((bn, 1), lambda j: (j, 0)),
        ],
        out_specs=pl.BlockSpec((bn, n_tok), lambda j: (j, 0)),
        out_shape=jax.ShapeDtypeStruct((n_out, n_tok), _F32),
    )(xt, wt, b)


def _tail_call(xt, p, w, b, bn, w_transposed):
    d, n_tok = xt.shape
    k = p.shape[1]
    n_out = w.shape[0] if w_transposed else w.shape[1]
    if w_transposed:
        w_spec = pl.BlockSpec((bn, k), lambda j: (j, 0))
    else:
        w_spec = pl.BlockSpec((k, bn), lambda j: (0, j))
    return pl.pallas_call(
        functools.partial(_tail_body, w_transposed=w_transposed),
        grid=(pl.cdiv(n_out, bn),),
        in_specs=[
            pl.BlockSpec((d, n_tok), lambda j: (0, 0)),
            pl.BlockSpec((d, k), lambda j: (0, 0)),
            w_spec,
            pl.BlockSpec((bn, 1), lambda j: (j, 0)),
        ],
        out_specs=pl.BlockSpec((bn, n_tok), lambda j: (j, 0)),
        out_shape=jax.ShapeDtypeStruct((n_out, n_tok), _F32),
        scratch_shapes=[pltpu.VMEM((k, n_tok), _BF)],
    )(xt, p, w, b)


def kernel(x, W0, b0, P1, W1, b1, P2, W2, b2):
    xt = _xt_call(x)
    lh = _head_call(xt, W0.T, b0.reshape(-1, 1), bn=512)
    lc1 = _tail_call(xt, P1, W1.T, b1.reshape(-1, 1), bn=512, w_transposed=True)
    lc2 = _tail_call(xt, P2, W2, b2.reshape(-1, 1), bn=1024, w_transposed=False)
    return (lh.T, lc1.T, lc2.T)


# speedup vs baseline: 3.1467x; 1.0218x over previous
import functools

import jax
import jax.numpy as jnp
from jax import lax
from jax.experimental import pallas as pl
from jax.experimental.pallas import tpu as pltpu

_BF = jnp.bfloat16
_F32 = jnp.float32


def _xt_body(x_ref, o_ref):
    o_ref[...] = x_ref[...].astype(_BF).T


def _xt_call(x):
    n_tok, d = x.shape
    return pl.pallas_call(
        _xt_body,
        out_shape=jax.ShapeDtypeStruct((d, n_tok), _BF),
    )(x)


def _head_body(xt_ref, wt_ref, b_ref, o_ref):
    o_ref[...] = (
        jnp.dot(wt_ref[...].astype(_BF), xt_ref[...], preferred_element_type=_F32)
        + b_ref[...]
    )


def _tail_body(xt_ref, p_ref, w_ref, b_ref, o_ref, h_ref, *, w_transposed):
    @pl.when(pl.program_id(0) == 0)
    def _():
        h_ref[...] = lax.dot_general(
            p_ref[...].astype(_BF),
            xt_ref[...],
            (((0,), (0,)), ((), ())),
            preferred_element_type=_F32,
        ).astype(_BF)

    if w_transposed:
        acc = jnp.dot(w_ref[...].astype(_BF), h_ref[...], preferred_element_type=_F32)
    else:
        acc = lax.dot_general(
            w_ref[...].astype(_BF),
            h_ref[...],
            (((0,), (0,)), ((), ())),
            preferred_element_type=_F32,
        )
    o_ref[...] = acc + b_ref[...]


def _head_call(xt, wt, b, bn):
    d, n_tok = xt.shape
    n_out = wt.shape[0]
    return pl.pallas_call(
        _head_body,
        grid=(pl.cdiv(n_out, bn),),
        in_specs=[
            pl.BlockSpec((d, n_tok), lambda j: (0, 0)),
            pl.BlockSpec((bn, d), lambda j: (j, 0)),
            pl.BlockSpec((bn, 1), lambda j: (j, 0)),
        ],
        out_specs=pl.BlockSpec((bn, n_tok), lambda j: (j, 0)),
        out_shape=jax.ShapeDtypeStruct((n_out, n_tok), _F32),
    )(xt, wt, b)


def _tail_call(xt, p, w, b, bn, w_transposed):
    d, n_tok = xt.shape
    k = p.shape[1]
    n_out = w.shape[0] if w_transposed else w.shape[1]
    if w_transposed:
        w_spec = pl.BlockSpec((bn, k), lambda j: (j, 0))
    else:
        w_spec = pl.BlockSpec((k, bn), lambda j: (0, j))
    return pl.pallas_call(
        functools.partial(_tail_body, w_transposed=w_transposed),
        grid=(pl.cdiv(n_out, bn),),
        in_specs=[
            pl.BlockSpec((d, n_tok), lambda j: (0, 0)),
            pl.BlockSpec((d, k), lambda j: (0, 0)),
            w_spec,
            pl.BlockSpec((bn, 1), lambda j: (j, 0)),
        ],
        out_specs=pl.BlockSpec((bn, n_tok), lambda j: (j, 0)),
        out_shape=jax.ShapeDtypeStruct((n_out, n_tok), _F32),
        scratch_shapes=[pltpu.VMEM((k, n_tok), _BF)],
    )(xt, p, w, b)


def kernel(x, W0, b0, P1, W1, b1, P2, W2, b2):
    xt = _xt_call(x)
    lh = _head_call(xt, W0.T, b0.reshape(-1, 1), bn=512)
    lc1 = _tail_call(xt, P1, W1.T, b1.reshape(-1, 1), bn=1000, w_transposed=True)
    lc2 = _tail_call(xt, P2, W2, b2.reshape(-1, 1), bn=2048, w_transposed=False)
    return (lh.T, lc1.T, lc2.T)
